# baseline (speedup 1.0000x reference)
import jax
import jax.numpy as jnp
import numpy as np
from jax import lax
from jax.experimental import pallas as pl
from jax.experimental.pallas import tpu as pltpu
from jax.experimental.pallas import tpu_sc as plsc

_N = 10000
_E = 320000
_F = 128
_CUTOFF = 5.0
_NUM_ATOMS = 20

_NC = 2
_NS = 16
_EB = 64
_NBATCH = 320
_G = 32
_D = 4
_SINK = _N
_NB = 1000


def _tc_body(ns_ref, nv_ref, dis_ref, w1_ref, b1_ref, w2a_ref, b2a_ref,
             w2c_ref, b2c_ref, wfa_ref, bfa_ref, wfc_ref, bfc_ref, rep_ref,
             q0_ref, q1_ref, q2_ref, q3_ref, v0_ref, v1_ref, v2_ref):
    d = dis_ref[...]
    n = lax.broadcasted_iota(jnp.int32, (d.shape[0], _NUM_ATOMS), 1
                             ).astype(jnp.float32) + 1.0
    rbf = jnp.sin(n * (np.pi / _CUTOFF) * d) / d
    cut = jnp.where(d <= _CUTOFF,
                    0.5 * (jnp.cos((np.pi / _CUTOFF) * d) + 1.0),
                    jnp.float32(0.0))
    z = jnp.dot(ns_ref[...], w1_ref[...],
                preferred_element_type=jnp.float32) + b1_ref[...]
    h = z * (1.0 / (1.0 + jnp.exp(-z)))
    sa = jnp.dot(h, w2a_ref[...], preferred_element_type=jnp.float32) + b2a_ref[...]
    sc = jnp.dot(h, w2c_ref[...], preferred_element_type=jnp.float32) + b2c_ref[...]
    fa = jnp.dot(rbf, wfa_ref[...], preferred_element_type=jnp.float32) + bfa_ref[...]
    fc = jnp.dot(rbf, wfc_ref[...], preferred_element_type=jnp.float32) + bfc_ref[...]
    g1 = fa * cut * sa
    g3 = fc * cut * sc
    q0_ref[...] = g3
    g1r = jnp.dot(g1, rep_ref[...], preferred_element_type=jnp.float32)
    nv = nv_ref[...]
    v = nv * g1r
    q1_ref[...] = v[:, 0:128]
    q2_ref[...] = v[:, 128:256]
    q3_ref[...] = v[:, 256:384]
    v0_ref[...] = nv[:, 0:128]
    v1_ref[...] = nv[:, 128:256]
    v2_ref[...] = nv[:, 256:384]


def _tc_payload(node_s, nv_flat, dis2d, w1, b1, w2a, b2a, w2c, b2c,
                wfa, bfa, wfc, bfc, rep):
    grid = (_N // _NB,)
    row = lambda i: (i, 0)
    full = lambda i: (0, 0)
    f32 = jnp.float32
    return pl.pallas_call(
        _tc_body,
        grid=grid,
        in_specs=[
            pl.BlockSpec((_NB, _F), row),
            pl.BlockSpec((_NB, 3 * _F), row),
            pl.BlockSpec((_NB, 1), row),
            pl.BlockSpec((_F, _F), full),
            pl.BlockSpec((1, _F), full),
            pl.BlockSpec((_F, _F), full),
            pl.BlockSpec((1, _F), full),
            pl.BlockSpec((_F, _F), full),
            pl.BlockSpec((1, _F), full),
            pl.BlockSpec((_NUM_ATOMS, _F), full),
            pl.BlockSpec((1, _F), full),
            pl.BlockSpec((_NUM_ATOMS, _F), full),
            pl.BlockSpec((1, _F), full),
            pl.BlockSpec((_F, 3 * _F), full),
        ],
        out_specs=[pl.BlockSpec((_NB, _F), row)] * 7,
        out_shape=[jax.ShapeDtypeStruct((_N, _F), f32)] * 7,
    )(node_s, nv_flat, dis2d, w1, b1, w2a, b2a, w2c, b2c, wfa, bfa, wfc, bfc,
      rep)


def _sc_body(q0, q1, q2, q3, src_h, dst_h, b0, b1, b2, b3,
             o0, o1, o2, o3, acc, idx_s, idx_d, bufs, sem):
    c = lax.axis_index("c")
    s = lax.axis_index("s")
    rpt = 624
    tail = _N - _NS * rpt
    r0 = s * rpt

    def do_chunk(q_hbm, base_hbm, out_hbm):
        pltpu.sync_copy(base_hbm.at[pl.ds(r0, rpt)], acc.at[pl.ds(r0, rpt)])

        @pl.when(s == 0)
        def _():
            pltpu.sync_copy(base_hbm.at[pl.ds(_NS * rpt, tail)],
                            acc.at[pl.ds(_NS * rpt, tail)])

        plsc.subcore_barrier()

        def group(g, carry):
            pltpu.sync_copy(src_h.at[s, pl.ds(g * _G, _G)], idx_s)
            pltpu.sync_copy(dst_h.at[s, pl.ds(g * _G, _G)], idx_d)
            for d in range(_D):
                pltpu.async_copy(q_hbm.at[idx_s.at[d]], bufs[d], sem)

            def pipe(i, carry2):
                for d in range(_D):
                    j = i * _D + d
                    pltpu.make_async_copy(q_hbm.at[idx_s.at[j]], bufs[d],
                                          sem).wait()
                    pltpu.sync_copy(bufs[d], acc.at[idx_d.at[j]], add=True)

                    @pl.when(j + _D < _G)
                    def _():
                        pltpu.async_copy(q_hbm.at[idx_s.at[j + _D]],
                                         bufs[d], sem)
                return carry2

            lax.fori_loop(0, _G // _D, pipe, 0)
            return carry

        lax.fori_loop(0, _NBATCH // _G, group, 0)
        plsc.subcore_barrier()
        pltpu.sync_copy(acc.at[pl.ds(r0, rpt)], out_hbm.at[pl.ds(r0, rpt)])

        @pl.when(s == 0)
        def _():
            pltpu.sync_copy(acc.at[pl.ds(_NS * rpt, tail)],
                            out_hbm.at[pl.ds(_NS * rpt, tail)])

        plsc.subcore_barrier()

    @pl.when(c == 0)
    def _():
        do_chunk(q0, b0, o0)
        do_chunk(q1, b1, o1)

    @pl.when(c == 1)
    def _():
        do_chunk(q2, b2, o2)
        do_chunk(q3, b3, o3)


def _sc_scatter(q0, q1, q2, q3, src, dst, b0, b1, b2, b3):
    f32 = jnp.float32
    mesh = plsc.VectorSubcoreMesh(core_axis_name="c", subcore_axis_name="s",
                                  num_cores=_NC, num_subcores=_NS)
    fn = pl.kernel(
        _sc_body,
        out_type=[jax.ShapeDtypeStruct((_N, _F), f32)] * 4,
        mesh=mesh,
        scratch_types=[
            pltpu.VMEM_SHARED((_N + 8, _F), f32),
            pltpu.VMEM((_G, _EB), jnp.int32),
            pltpu.VMEM((_G, _EB), jnp.int32),
            [pltpu.VMEM((_EB, _F), f32) for _ in range(_D)],
            pltpu.SemaphoreType.DMA,
        ],
    )
    return fn(q0, q1, q2, q3, src, dst, b0, b1, b2, b3)


def _repeat3_matrix():
    r = np.zeros((_F, 3 * _F), dtype=np.float32)
    for f in range(_F):
        r[f, 3 * f:3 * f + 3] = 1.0
    return jnp.asarray(r)


@jax.jit
def kernel(node_s, node_vec, edge, edge_difference, edge_dis,
           W1, b1, W2, b2, Wf, bf):
    del edge_difference
    nv_flat = node_vec.reshape(_N, 3 * _F)
    dis2d = edge_dis.reshape(_N, 1)
    pad = _NS * _NBATCH * _EB - _E
    src = jnp.concatenate(
        [edge[:, 1], jnp.zeros((pad,), jnp.int32)]).reshape(
            _NS, _NBATCH, _EB)
    dst = jnp.concatenate(
        [edge[:, 0], jnp.full((pad,), _SINK, jnp.int32)]).reshape(
            _NS, _NBATCH, _EB)
    w2a, b2a = W2[:, :_F], b2[:_F].reshape(1, _F)
    w2c, b2c = W2[:, 2 * _F:], b2[2 * _F:].reshape(1, _F)
    wfa, bfa = Wf[:, :_F], bf[:_F].reshape(1, _F)
    wfc, bfc = Wf[:, 2 * _F:], bf[2 * _F:].reshape(1, _F)
    rep = _repeat3_matrix()

    q0, q1, q2, q3, v0, v1, v2 = _tc_payload(
        node_s, nv_flat, dis2d, W1, b1.reshape(1, _F), w2a, b2a, w2c, b2c,
        wfa, bfa, wfc, bfc, rep)

    o0, o1, o2, o3 = _sc_scatter(q0, q1, q2, q3, src, dst,
                                 node_s, v0, v1, v2)

    delta_node_scalar = o0
    delta_node_vector = jnp.concatenate([o1, o2, o3], axis=1).reshape(
        _N, _F, 3)
    return (delta_node_vector, delta_node_scalar)


# speedup vs baseline: 1.5555x; 1.5555x over previous
import jax
import jax.numpy as jnp
import numpy as np
from jax import lax
from jax.experimental import pallas as pl
from jax.experimental.pallas import tpu as pltpu
from jax.experimental.pallas import tpu_sc as plsc

_N = 10000
_E = 320000
_F = 128
_CUTOFF = 5.0
_NUM_ATOMS = 20

_NC = 2
_NS = 16
_C = 64
_NCHUNK = 8
_EB = 128
_NBATCH = 160
_G = 32
_D = 2
_SINK = _N
_NB = 1000


def _tc_body(*refs):
    (ns_ref, nv_ref, dis_ref, w1_ref, b1_ref, w2a_ref, b2a_ref,
     w2c_ref, b2c_ref, wfa_ref, bfa_ref, wfc_ref, bfc_ref, rep_ref) = refs[:14]
    out = refs[14:]
    d = dis_ref[...]
    n = lax.broadcasted_iota(jnp.int32, (d.shape[0], _NUM_ATOMS), 1
                             ).astype(jnp.float32) + 1.0
    rbf = jnp.sin(n * (np.pi / _CUTOFF) * d) / d
    cut = jnp.where(d <= _CUTOFF,
                    0.5 * (jnp.cos((np.pi / _CUTOFF) * d) + 1.0),
                    jnp.float32(0.0))
    z = jnp.dot(ns_ref[...], w1_ref[...],
                preferred_element_type=jnp.float32) + b1_ref[...]
    h = z * (1.0 / (1.0 + jnp.exp(-z)))
    sa = jnp.dot(h, w2a_ref[...], preferred_element_type=jnp.float32) + b2a_ref[...]
    sc = jnp.dot(h, w2c_ref[...], preferred_element_type=jnp.float32) + b2c_ref[...]
    fa = jnp.dot(rbf, wfa_ref[...], preferred_element_type=jnp.float32) + bfa_ref[...]
    fc = jnp.dot(rbf, wfc_ref[...], preferred_element_type=jnp.float32) + bfc_ref[...]
    g1 = fa * cut * sa
    g3 = fc * cut * sc
    g1r = jnp.dot(g1, rep_ref[...], preferred_element_type=jnp.float32)
    nv = nv_ref[...]
    v = nv * g1r
    pay = [g3[:, :_C], g3[:, _C:]] + [
        v[:, i * _C:(i + 1) * _C] for i in range(6)]
    base = [ns_ref[:, :_C], ns_ref[:, _C:]] + [
        nv[:, i * _C:(i + 1) * _C] for i in range(6)]
    for k in range(_NCHUNK):
        out[k][...] = pay[k]
        out[_NCHUNK + k][...] = base[k]


def _tc_payload(node_s, nv_flat, dis2d, w1, b1, w2a, b2a, w2c, b2c,
                wfa, bfa, wfc, bfc, rep):
    grid = (_N // _NB,)
    row = lambda i: (i, 0)
    full = lambda i: (0, 0)
    f32 = jnp.float32
    return pl.pallas_call(
        _tc_body,
        grid=grid,
        in_specs=[
            pl.BlockSpec((_NB, _F), row),
            pl.BlockSpec((_NB, 3 * _F), row),
            pl.BlockSpec((_NB, 1), row),
            pl.BlockSpec((_F, _F), full),
            pl.BlockSpec((1, _F), full),
            pl.BlockSpec((_F, _F), full),
            pl.BlockSpec((1, _F), full),
            pl.BlockSpec((_F, _F), full),
            pl.BlockSpec((1, _F), full),
            pl.BlockSpec((_NUM_ATOMS, _F), full),
            pl.BlockSpec((1, _F), full),
            pl.BlockSpec((_NUM_ATOMS, _F), full),
            pl.BlockSpec((1, _F), full),
            pl.BlockSpec((_F, 3 * _F), full),
        ],
        out_specs=[pl.BlockSpec((_NB, _C), row)] * (2 * _NCHUNK),
        out_shape=[jax.ShapeDtypeStruct((_N, _C), f32)] * (2 * _NCHUNK),
    )(node_s, nv_flat, dis2d, w1, b1, w2a, b2a, w2c, b2c, wfa, bfa, wfc, bfc,
      rep)


def _sc_body(*refs):
    qs = refs[:_NCHUNK]
    src_h, dst_h = refs[_NCHUNK], refs[_NCHUNK + 1]
    bs = refs[_NCHUNK + 2:2 * _NCHUNK + 2]
    os = refs[2 * _NCHUNK + 2:3 * _NCHUNK + 2]
    table, acc, idx_s, idx_d, bufs, sem = refs[3 * _NCHUNK + 2:]
    c = lax.axis_index("c")
    s = lax.axis_index("s")
    rpt = 624
    tail = _N - _NS * rpt
    r0 = s * rpt

    def do_chunk(q_hbm, base_hbm, out_hbm):
        pltpu.sync_copy(q_hbm.at[pl.ds(r0, rpt)], table.at[pl.ds(r0, rpt)])
        pltpu.sync_copy(base_hbm.at[pl.ds(r0, rpt)], acc.at[pl.ds(r0, rpt)])

        @pl.when(s == 0)
        def _():
            pltpu.sync_copy(q_hbm.at[pl.ds(_NS * rpt, tail)],
                            table.at[pl.ds(_NS * rpt, tail)])
            pltpu.sync_copy(base_hbm.at[pl.ds(_NS * rpt, tail)],
                            acc.at[pl.ds(_NS * rpt, tail)])

        plsc.subcore_barrier()

        def group(g, carry):
            pltpu.sync_copy(src_h.at[s, pl.ds(g * _G, _G)], idx_s)
            pltpu.sync_copy(dst_h.at[s, pl.ds(g * _G, _G)], idx_d)
            for d in range(_D):
                pltpu.async_copy(table.at[idx_s.at[d]], bufs[d], sem)

            def pipe(i, carry2):
                for d in range(_D):
                    j = i * _D + d
                    pltpu.make_async_copy(table.at[idx_s.at[j]], bufs[d],
                                          sem).wait()
                    pltpu.sync_copy(bufs[d], acc.at[idx_d.at[j]], add=True)

                    @pl.when(j + _D < _G)
                    def _():
                        pltpu.async_copy(table.at[idx_s.at[j + _D]],
                                         bufs[d], sem)
                return carry2

            lax.fori_loop(0, _G // _D, pipe, 0)
            return carry

        lax.fori_loop(0, _NBATCH // _G, group, 0)
        plsc.subcore_barrier()
        pltpu.sync_copy(acc.at[pl.ds(r0, rpt)], out_hbm.at[pl.ds(r0, rpt)])

        @pl.when(s == 0)
        def _():
            pltpu.sync_copy(acc.at[pl.ds(_NS * rpt, tail)],
                            out_hbm.at[pl.ds(_NS * rpt, tail)])

        plsc.subcore_barrier()

    @pl.when(c == 0)
    def _():
        for k in range(4):
            do_chunk(qs[k], bs[k], os[k])

    @pl.when(c == 1)
    def _():
        for k in range(4, 8):
            do_chunk(qs[k], bs[k], os[k])


def _sc_scatter(qs, src, dst, bs):
    f32 = jnp.float32
    mesh = plsc.VectorSubcoreMesh(core_axis_name="c", subcore_axis_name="s",
                                  num_cores=_NC, num_subcores=_NS)
    fn = pl.kernel(
        _sc_body,
        out_type=[jax.ShapeDtypeStruct((_N, _C), f32)] * _NCHUNK,
        mesh=mesh,
        scratch_types=[
            pltpu.VMEM_SHARED((_N, _C), f32),
            pltpu.VMEM_SHARED((_N + 8, _C), f32),
            pltpu.VMEM((_G, _EB), jnp.int32),
            pltpu.VMEM((_G, _EB), jnp.int32),
            [pltpu.VMEM((_EB, _C), f32) for _ in range(_D)],
            pltpu.SemaphoreType.DMA,
        ],
    )
    return fn(*qs, src, dst, *bs)


def _repeat3_matrix():
    r = np.zeros((_F, 3 * _F), dtype=np.float32)
    for f in range(_F):
        r[f, 3 * f:3 * f + 3] = 1.0
    return jnp.asarray(r)


@jax.jit
def kernel(node_s, node_vec, edge, edge_difference, edge_dis,
           W1, b1, W2, b2, Wf, bf):
    del edge_difference
    nv_flat = node_vec.reshape(_N, 3 * _F)
    dis2d = edge_dis.reshape(_N, 1)
    pad = _NS * _NBATCH * _EB - _E
    src = jnp.concatenate(
        [edge[:, 1], jnp.zeros((pad,), jnp.int32)]).reshape(
            _NS, _NBATCH, _EB)
    dst = jnp.concatenate(
        [edge[:, 0], jnp.full((pad,), _SINK, jnp.int32)]).reshape(
            _NS, _NBATCH, _EB)
    w2a, b2a = W2[:, :_F], b2[:_F].reshape(1, _F)
    w2c, b2c = W2[:, 2 * _F:], b2[2 * _F:].reshape(1, _F)
    wfa, bfa = Wf[:, :_F], bf[:_F].reshape(1, _F)
    wfc, bfc = Wf[:, 2 * _F:], bf[2 * _F:].reshape(1, _F)
    rep = _repeat3_matrix()

    outs = _tc_payload(
        node_s, nv_flat, dis2d, W1, b1.reshape(1, _F), w2a, b2a, w2c, b2c,
        wfa, bfa, wfc, bfc, rep)
    qs, bs = outs[:_NCHUNK], outs[_NCHUNK:]

    os = _sc_scatter(qs, src, dst, bs)

    delta_node_scalar = jnp.concatenate(os[:2], axis=1)
    delta_node_vector = jnp.concatenate(os[2:], axis=1).reshape(_N, _F, 3)
    return (delta_node_vector, delta_node_scalar)
